# baseline (speedup 1.0000x reference)
import jax
import jax.numpy as jnp
from jax.experimental import pallas as pl
from jax.experimental.pallas import tpu as pltpu
from jax.experimental.pallas import tpu_sc as plsc

_BLK = 16
_LANES = 16


def kernel(x, pos_table):
    B, S, D = x.shape
    mesh = plsc.VectorSubcoreMesh(core_axis_name="c", subcore_axis_name="s")

    @pl.kernel(out_type=jax.ShapeDtypeStruct((B, S, D), x.dtype), mesh=mesh)
    def sc_kernel(x_hbm, p_hbm, o_hbm):
        def body(x_vmem, p_vmem, o_vmem):
            @pl.loop(0, _BLK)
            def _(r):
                @pl.loop(0, D, step=_LANES)
                def _(c):
                    o_vmem.at[0, r, pl.ds(c, _LANES)][...] = (
                        x_vmem.at[0, r, pl.ds(c, _LANES)][...]
                        + p_vmem.at[r, pl.ds(c, _LANES)][...]
                    )

        pltpu.emit_pipeline(
            body,
            grid=(B, S // _BLK),
            in_specs=[
                pl.BlockSpec((1, _BLK, D), index_map=lambda b, i: (b, i, 0)),
                pl.BlockSpec((_BLK, D), index_map=lambda b, i: (i, 0)),
            ],
            out_specs=[pl.BlockSpec((1, _BLK, D), index_map=lambda b, i: (b, i, 0))],
            core_axis_name=("c", "s"),
            dimension_semantics=(pltpu.PARALLEL, pltpu.PARALLEL),
        )(x_hbm, p_hbm, o_hbm)

    return sc_kernel(x, pos_table)


# speedup vs baseline: 4.3903x; 4.3903x over previous
import jax
import jax.numpy as jnp
from jax.experimental import pallas as pl

_S_BLK = 512


def _add_body(x_ref, p_ref, o_ref):
    o_ref[...] = x_ref[...] + p_ref[...][None, :, :]


def kernel(x, pos_table):
    B, S, D = x.shape
    grid = (S // _S_BLK,)
    return pl.pallas_call(
        _add_body,
        grid=grid,
        in_specs=[
            pl.BlockSpec((B, _S_BLK, D), lambda i: (0, i, 0)),
            pl.BlockSpec((_S_BLK, D), lambda i: (i, 0)),
        ],
        out_specs=pl.BlockSpec((B, _S_BLK, D), lambda i: (0, i, 0)),
        out_shape=jax.ShapeDtypeStruct((B, S, D), x.dtype),
    )(x, pos_table)
